# baseline (speedup 1.0000x reference)
import functools

import jax
import jax.numpy as jnp
from jax import lax
from jax.experimental import pallas as pl
from jax.experimental.pallas import tpu as pltpu
from jax.experimental.pallas import tpu_sc as plsc

NC = 2
NS = 16
L = 16
NW = NC * NS

ROWS, COLS = 128, 32768
TOTAL = ROWS * COLS
PER_W = TOTAL // NW
CHUNK = 32768
NCHUNK = PER_W // CHUNK
UNROLL = 8
VEC_ITERS = CHUNK // (L * UNROLL)

_mesh = plsc.VectorSubcoreMesh(core_axis_name="c", subcore_axis_name="s")


@functools.partial(
    pl.kernel,
    out_type=jax.ShapeDtypeStruct((NW, 2, L), jnp.int32),
    mesh=_mesh,
    scratch_types=[
        pltpu.VMEM((CHUNK,), jnp.float32),
        pltpu.VMEM((CHUNK,), jnp.float32),
        pltpu.VMEM((2, L), jnp.int32),
        pltpu.SemaphoreType.DMA,
        pltpu.SemaphoreType.DMA,
    ],
)
def _count_nonneg(x_hbm, y_hbm, out_hbm, buf0, buf1, acc_v, sem0, sem1):
    wid = lax.axis_index("s") * NC + lax.axis_index("c")
    base = wid * PER_W

    bufs = (buf0, buf1)
    sems = (sem0, sem1)
    srcs = [x_hbm.at[pl.ds(base + j * CHUNK, CHUNK)] for j in range(NCHUNK)]
    srcs += [y_hbm.at[pl.ds(base + j * CHUNK, CHUNK)] for j in range(NCHUNK)]
    copies = [
        pltpu.make_async_copy(srcs[j], bufs[j % 2], sems[j % 2])
        for j in range(2 * NCHUNK)
    ]

    def chunk_count(buf, acc):
        def it(i, a):
            for u in range(UNROLL):
                v = buf[pl.ds((i * UNROLL + u) * L, L)]
                a = a + jnp.where(v >= 0.0, 1, 0).astype(jnp.int32)
            return a
        return lax.fori_loop(0, VEC_ITERS, it, acc)

    copies[0].start()
    accs = [jnp.zeros((L,), jnp.int32), jnp.zeros((L,), jnp.int32)]
    for j in range(2 * NCHUNK):
        if j + 1 < 2 * NCHUNK:
            copies[j + 1].start()
        copies[j].wait()
        accs[j // NCHUNK] = chunk_count(bufs[j % 2], accs[j // NCHUNK])

    acc_v[0, :] = accs[0]
    acc_v[1, :] = accs[1]
    pltpu.sync_copy(acc_v, out_hbm.at[wid])


def kernel(x, y):
    partials = _count_nonneg(x.reshape(TOTAL), y.reshape(TOTAL))
    sums = jnp.sum(partials, axis=(0, 2), dtype=jnp.int32)
    return (sums[0], sums[1])


# speedup vs baseline: 1.7220x; 1.7220x over previous
import functools

import jax
import jax.numpy as jnp
from jax import lax
from jax.experimental import pallas as pl
from jax.experimental.pallas import tpu as pltpu
from jax.experimental.pallas import tpu_sc as plsc

NC = 2
NS = 16
L = 16
NW = NC * NS

ROWS, COLS = 128, 32768
TOTAL = ROWS * COLS
STRIPE_R = 8
COL_HALF = COLS // 2
CHUNK_C = 4096
NCHUNK = COL_HALF // CHUNK_C
VEC_ITERS = CHUNK_C // L

_mesh = plsc.VectorSubcoreMesh(core_axis_name="c", subcore_axis_name="s")


@functools.partial(
    pl.kernel,
    out_type=jax.ShapeDtypeStruct((NW, 2, L), jnp.int32),
    mesh=_mesh,
    scratch_types=[
        pltpu.VMEM((STRIPE_R, CHUNK_C), jnp.float32),
        pltpu.VMEM((STRIPE_R, CHUNK_C), jnp.float32),
        pltpu.VMEM((2, L), jnp.int32),
        pltpu.SemaphoreType.DMA,
        pltpu.SemaphoreType.DMA,
    ],
    compiler_params=pltpu.CompilerParams(
        use_tc_tiling_on_sc=True, needs_layout_passes=False
    ),
)
def _count_neg(x_hbm, y_hbm, out_hbm, buf0, buf1, acc_v, sem0, sem1):
    wid = lax.axis_index("s") * NC + lax.axis_index("c")
    row0 = (wid % NS) * STRIPE_R
    col0 = (wid // NS) * COL_HALF

    bufs = (buf0, buf1)
    sems = (sem0, sem1)
    srcs = [
        src.at[pl.ds(row0, STRIPE_R), pl.ds(col0 + j * CHUNK_C, CHUNK_C)]
        for src in (x_hbm, y_hbm)
        for j in range(NCHUNK)
    ]
    copies = [
        pltpu.make_async_copy(srcs[j], bufs[j % 2], sems[j % 2])
        for j in range(2 * NCHUNK)
    ]

    def chunk_count(buf, acc):
        def it(i, a):
            c = i * L
            for r in range(STRIPE_R):
                vi = plsc.bitcast(buf[r, pl.ds(c, L)], jnp.int32)
                a = a + lax.shift_right_arithmetic(vi, 31)
            return a
        return lax.fori_loop(0, VEC_ITERS, it, acc)

    copies[0].start()
    accs = [jnp.zeros((L,), jnp.int32), jnp.zeros((L,), jnp.int32)]
    for j in range(2 * NCHUNK):
        if j + 1 < 2 * NCHUNK:
            copies[j + 1].start()
        copies[j].wait()
        accs[j // NCHUNK] = chunk_count(bufs[j % 2], accs[j // NCHUNK])

    acc_v[0, :] = accs[0]
    acc_v[1, :] = accs[1]
    pltpu.sync_copy(acc_v, out_hbm.at[wid])


def kernel(x, y):
    partials = _count_neg(x, y)
    neg = jnp.sum(partials, axis=(0, 2), dtype=jnp.int32)
    return (TOTAL + neg[0], TOTAL + neg[1])


# speedup vs baseline: 1.8713x; 1.0867x over previous
import functools

import jax
import jax.numpy as jnp
from jax import lax
from jax.experimental import pallas as pl
from jax.experimental.pallas import tpu as pltpu
from jax.experimental.pallas import tpu_sc as plsc

NC = 2
NS = 16
L = 16
NW = NC * NS

ROWS, COLS = 128, 32768
TOTAL = ROWS * COLS
SPLIT_R = 64

SC_ROWS = ROWS - SPLIT_R
STRIPE_R = 8
N_STRIPES = SC_ROWS // STRIPE_R
COL_SPLIT = NW // N_STRIPES
WCOLS = COLS // COL_SPLIT
CHUNK_C = 4096
NCHUNK = WCOLS // CHUNK_C
VEC_ITERS = CHUNK_C // L

_mesh = plsc.VectorSubcoreMesh(core_axis_name="c", subcore_axis_name="s")


@functools.partial(
    pl.kernel,
    out_type=jax.ShapeDtypeStruct((NW, 2, L), jnp.int32),
    mesh=_mesh,
    scratch_types=[
        pltpu.VMEM((STRIPE_R, CHUNK_C), jnp.float32),
        pltpu.VMEM((STRIPE_R, CHUNK_C), jnp.float32),
        pltpu.VMEM((2, L), jnp.int32),
        pltpu.SemaphoreType.DMA,
        pltpu.SemaphoreType.DMA,
    ],
    compiler_params=pltpu.CompilerParams(
        use_tc_tiling_on_sc=True, needs_layout_passes=False
    ),
)
def _sc_count_neg(x_hbm, y_hbm, out_hbm, buf0, buf1, acc_v, sem0, sem1):
    wid = lax.axis_index("s") * NC + lax.axis_index("c")
    row0 = SPLIT_R + (wid % N_STRIPES) * STRIPE_R
    col0 = (wid // N_STRIPES) * WCOLS

    bufs = (buf0, buf1)
    sems = (sem0, sem1)
    srcs = [
        src.at[pl.ds(row0, STRIPE_R), pl.ds(col0 + j * CHUNK_C, CHUNK_C)]
        for src in (x_hbm, y_hbm)
        for j in range(NCHUNK)
    ]
    copies = [
        pltpu.make_async_copy(srcs[j], bufs[j % 2], sems[j % 2])
        for j in range(2 * NCHUNK)
    ]

    def chunk_count(buf, acc):
        def it(i, a):
            c = i * L
            for r in range(STRIPE_R):
                vi = plsc.bitcast(buf[r, pl.ds(c, L)], jnp.int32)
                a = a + lax.shift_right_arithmetic(vi, 31)
            return a
        return lax.fori_loop(0, VEC_ITERS, it, acc)

    copies[0].start()
    accs = [jnp.zeros((L,), jnp.int32), jnp.zeros((L,), jnp.int32)]
    for j in range(2 * NCHUNK):
        if j + 1 < 2 * NCHUNK:
            copies[j + 1].start()
        copies[j].wait()
        accs[j // NCHUNK] = chunk_count(bufs[j % 2], accs[j // NCHUNK])

    acc_v[0, :] = accs[0]
    acc_v[1, :] = accs[1]
    pltpu.sync_copy(acc_v, out_hbm.at[wid])


TC_BLK_C = 4096
TC_GRID = COLS // TC_BLK_C


def _tc_body(x_ref, y_ref, a_ref, c_ref):
    @pl.when(pl.program_id(0) == 0)
    def _():
        a_ref[0, 0] = 0
        c_ref[0, 0] = 0

    a_ref[0, 0] += jnp.sum(x_ref[...] < 0.0).astype(jnp.int32)
    c_ref[0, 0] += jnp.sum(y_ref[...] < 0.0).astype(jnp.int32)


_tc_count_neg = pl.pallas_call(
    _tc_body,
    grid=(TC_GRID,),
    in_specs=[
        pl.BlockSpec((SPLIT_R, TC_BLK_C), lambda i: (0, i)),
        pl.BlockSpec((SPLIT_R, TC_BLK_C), lambda i: (0, i)),
    ],
    out_specs=[
        pl.BlockSpec(memory_space=pltpu.SMEM),
        pl.BlockSpec(memory_space=pltpu.SMEM),
    ],
    out_shape=[
        jax.ShapeDtypeStruct((1, 1), jnp.int32),
        jax.ShapeDtypeStruct((1, 1), jnp.int32),
    ],
)


def kernel(x, y):
    sc_part = _sc_count_neg(x, y)
    tc_a, tc_c = _tc_count_neg(x, y)
    sc_sum = jnp.sum(sc_part, axis=(0, 2), dtype=jnp.int32)
    a = TOTAL + sc_sum[0] - tc_a[0, 0]
    c = TOTAL + sc_sum[1] - tc_c[0, 0]
    return (a, c)


# speedup vs baseline: 1.8840x; 1.0067x over previous
import functools

import jax
import jax.numpy as jnp
from jax import lax
from jax.experimental import pallas as pl
from jax.experimental.pallas import tpu as pltpu
from jax.experimental.pallas import tpu_sc as plsc

NC = 2
NS = 16
L = 16
NW = NC * NS

ROWS, COLS = 128, 32768
TOTAL = ROWS * COLS
SPLIT_R = 64

SC_ROWS = ROWS - SPLIT_R
STRIPE_R = 8
N_STRIPES = SC_ROWS // STRIPE_R
COL_SPLIT = NW // N_STRIPES
WCOLS = COLS // COL_SPLIT
CHUNK_C = 4096
NCHUNK = WCOLS // CHUNK_C
VEC_ITERS = CHUNK_C // L

_mesh = plsc.VectorSubcoreMesh(core_axis_name="c", subcore_axis_name="s")


@functools.partial(
    pl.kernel,
    out_type=jax.ShapeDtypeStruct((NW, 2, L), jnp.int32),
    mesh=_mesh,
    scratch_types=[
        pltpu.VMEM((STRIPE_R, CHUNK_C), jnp.float32),
        pltpu.VMEM((STRIPE_R, CHUNK_C), jnp.float32),
        pltpu.VMEM((2, L), jnp.int32),
        pltpu.SemaphoreType.DMA,
        pltpu.SemaphoreType.DMA,
    ],
    compiler_params=pltpu.CompilerParams(
        use_tc_tiling_on_sc=True,
        needs_layout_passes=False,
        skip_device_barrier=True,
    ),
)
def _sc_count_neg(x_hbm, y_hbm, out_hbm, buf0, buf1, acc_v, sem0, sem1):
    wid = lax.axis_index("s") * NC + lax.axis_index("c")
    row0 = SPLIT_R + (wid % N_STRIPES) * STRIPE_R
    col0 = (wid // N_STRIPES) * WCOLS

    bufs = (buf0, buf1)
    sems = (sem0, sem1)
    srcs = [
        src.at[pl.ds(row0, STRIPE_R), pl.ds(col0 + j * CHUNK_C, CHUNK_C)]
        for src in (x_hbm, y_hbm)
        for j in range(NCHUNK)
    ]
    copies = [
        pltpu.make_async_copy(srcs[j], bufs[j % 2], sems[j % 2])
        for j in range(2 * NCHUNK)
    ]

    def chunk_count(buf, acc):
        def it(i, a):
            c = i * L
            for r in range(STRIPE_R):
                vi = plsc.bitcast(buf[r, pl.ds(c, L)], jnp.int32)
                a = a + lax.shift_right_arithmetic(vi, 31)
            return a
        return lax.fori_loop(0, VEC_ITERS, it, acc)

    copies[0].start()
    accs = [jnp.zeros((L,), jnp.int32), jnp.zeros((L,), jnp.int32)]
    for j in range(2 * NCHUNK):
        if j + 1 < 2 * NCHUNK:
            copies[j + 1].start()
        copies[j].wait()
        accs[j // NCHUNK] = chunk_count(bufs[j % 2], accs[j // NCHUNK])

    acc_v[0, :] = accs[0]
    acc_v[1, :] = accs[1]
    pltpu.sync_copy(acc_v, out_hbm.at[wid])


TC_BLK_C = 4096
TC_GRID = COLS // TC_BLK_C


def _tc_body(x_ref, y_ref, a_ref, c_ref):
    @pl.when(pl.program_id(0) == 0)
    def _():
        a_ref[0, 0] = 0
        c_ref[0, 0] = 0

    a_ref[0, 0] += jnp.sum(x_ref[...] < 0.0).astype(jnp.int32)
    c_ref[0, 0] += jnp.sum(y_ref[...] < 0.0).astype(jnp.int32)


_tc_count_neg = pl.pallas_call(
    _tc_body,
    grid=(TC_GRID,),
    in_specs=[
        pl.BlockSpec((SPLIT_R, TC_BLK_C), lambda i: (0, i)),
        pl.BlockSpec((SPLIT_R, TC_BLK_C), lambda i: (0, i)),
    ],
    out_specs=[
        pl.BlockSpec(memory_space=pltpu.SMEM),
        pl.BlockSpec(memory_space=pltpu.SMEM),
    ],
    out_shape=[
        jax.ShapeDtypeStruct((1, 1), jnp.int32),
        jax.ShapeDtypeStruct((1, 1), jnp.int32),
    ],
)


def kernel(x, y):
    sc_part = _sc_count_neg(x, y)
    tc_a, tc_c = _tc_count_neg(x, y)
    sc_sum = jnp.sum(sc_part, axis=(0, 2), dtype=jnp.int32)
    a = TOTAL + sc_sum[0] - tc_a[0, 0]
    c = TOTAL + sc_sum[1] - tc_c[0, 0]
    return (a, c)
